# Initial kernel scaffold; baseline (speedup 1.0000x reference)
import jax
import jax.numpy as jnp
from jax.experimental import pallas as pl


def kernel(input, values, bias, row_idx, col_idx):
    raise NotImplementedError("write your pallas kernel here")


# speedup vs baseline: 3.0198x; 3.0198x over previous
import functools

import jax
import jax.numpy as jnp
from jax import lax
from jax.experimental import pallas as pl
from jax.experimental.pallas import tpu as pltpu
from jax.experimental.pallas import tpu_sc as plsc

B = 256
IN_F = 4096
OUT_F = 4096

NC = 2
NS = 16
NW = NC * NS
BPW = B // NW

EDGE_CHUNK = 2048
LANES = 16


@functools.lru_cache(maxsize=None)
def _build(n_chunks: int):
    nnz_pad = n_chunks * EDGE_CHUNK
    mesh = plsc.VectorSubcoreMesh(core_axis_name="c", subcore_axis_name="s")

    @functools.partial(
        pl.kernel,
        mesh=mesh,
        out_type=jax.ShapeDtypeStruct((B, OUT_F), jnp.float32),
        compiler_params=pltpu.CompilerParams(needs_layout_passes=False),
        scratch_types=[
            pltpu.VMEM((BPW * IN_F,), jnp.float32),
            pltpu.VMEM((BPW * OUT_F,), jnp.float32),
            pltpu.VMEM((2 * EDGE_CHUNK,), jnp.int32),
            pltpu.VMEM((2 * EDGE_CHUNK,), jnp.int32),
            pltpu.VMEM((2 * EDGE_CHUNK,), jnp.float32),
            pltpu.SemaphoreType.DMA,
        ],
    )
    def spmm(x_hbm, val_hbm, bias_hbm, row_hbm, col_hbm, out_hbm,
             x_v, acc_v, row_v, col_v, val_v, sem):
        wid = lax.axis_index("s") * NC + lax.axis_index("c")
        b0 = wid * BPW

        for b in range(BPW):
            pltpu.sync_copy(x_hbm.at[b0 + b], x_v.at[pl.ds(b * IN_F, IN_F)])
            pltpu.sync_copy(bias_hbm, acc_v.at[pl.ds(b * OUT_F, OUT_F)])

        def fire(chunk, buf):
            off = chunk * EDGE_CHUNK
            dst = pl.ds(buf * EDGE_CHUNK, EDGE_CHUNK)
            pltpu.async_copy(row_hbm.at[pl.ds(off, EDGE_CHUNK)], row_v.at[dst], sem)
            pltpu.async_copy(col_hbm.at[pl.ds(off, EDGE_CHUNK)], col_v.at[dst], sem)
            pltpu.async_copy(val_hbm.at[pl.ds(off, EDGE_CHUNK)], val_v.at[dst], sem)

        def wait(buf):
            dst = pl.ds(buf * EDGE_CHUNK, EDGE_CHUNK)
            pltpu.make_async_copy(row_hbm.at[pl.ds(0, EDGE_CHUNK)], row_v.at[dst], sem).wait()
            pltpu.make_async_copy(col_hbm.at[pl.ds(0, EDGE_CHUNK)], col_v.at[dst], sem).wait()
            pltpu.make_async_copy(val_hbm.at[pl.ds(0, EDGE_CHUNK)], val_v.at[dst], sem).wait()

        def compute(buf):
            base = buf * EDGE_CHUNK

            def group(g, carry):
                r16 = row_v[pl.ds(base + g * LANES, LANES)]
                c16 = col_v[pl.ds(base + g * LANES, LANES)]
                v16 = val_v[pl.ds(base + g * LANES, LANES)]
                for b in range(BPW):
                    xv = plsc.load_gather(x_v, [c16 + (b * IN_F)])
                    plsc.addupdate_scatter(acc_v, [r16 + (b * OUT_F)], xv * v16)
                return carry

            lax.fori_loop(0, EDGE_CHUNK // LANES, group, 0)

        fire(0, 0)
        fire(1, 1)

        def step(k, carry):
            for buf in range(2):
                wait(buf)
                compute(buf)
                fire(2 * k + 2 + buf, buf)
            return carry

        lax.fori_loop(0, n_chunks // 2 - 1, step, 0)
        for buf in range(2):
            wait(buf)
            compute(buf)

        for b in range(BPW):
            pltpu.sync_copy(acc_v.at[pl.ds(b * OUT_F, OUT_F)], out_hbm.at[b0 + b])

    return spmm


def kernel(input, values, bias, row_idx, col_idx):
    nnz = values.shape[0]
    n_chunks = -(-nnz // EDGE_CHUNK)
    if n_chunks % 2:
        n_chunks += 1
    pad = n_chunks * EDGE_CHUNK - nnz
    if pad:
        row_idx = jnp.concatenate([row_idx, jnp.zeros((pad,), row_idx.dtype)])
        col_idx = jnp.concatenate([col_idx, jnp.zeros((pad,), col_idx.dtype)])
        values = jnp.concatenate([values, jnp.zeros((pad,), values.dtype)])
    return _build(n_chunks)(input, values, bias, row_idx, col_idx)


# speedup vs baseline: 8.3250x; 2.7568x over previous
import functools

import jax
import jax.numpy as jnp
from jax import lax
from jax.experimental import pallas as pl
from jax.experimental.pallas import tpu as pltpu
from jax.experimental.pallas import tpu_sc as plsc

B = 256
IN_F = 4096
OUT_F = 4096

NC = 2
NS = 16
NW = NC * NS
BPW = B // NW
PPW = BPW // 2

EDGE_CHUNK = 2048
CHUNK_WORDS = 2 * EDGE_CHUNK
LANES = 16


@functools.lru_cache(maxsize=None)
def _build(n_chunks: int):
    mesh = plsc.VectorSubcoreMesh(core_axis_name="c", subcore_axis_name="s")

    @functools.partial(
        pl.kernel,
        mesh=mesh,
        out_type=jax.ShapeDtypeStruct((B, OUT_F), jnp.float32),
        compiler_params=pltpu.CompilerParams(needs_layout_passes=False),
        scratch_types=[
            pltpu.VMEM((PPW * IN_F,), jnp.int32),
            pltpu.VMEM((BPW * OUT_F,), jnp.float32),
            pltpu.VMEM((2 * CHUNK_WORDS + LANES,), jnp.int32),
            pltpu.SemaphoreType.DMA,
        ],
    )
    def spmm(xpk_hbm, edges_hbm, bias_hbm, out_hbm, x_v, acc_v, edge_v, sem):
        wid = lax.axis_index("s") * NC + lax.axis_index("c")
        b0 = wid * BPW
        p0 = wid * PPW

        for p in range(PPW):
            pltpu.sync_copy(xpk_hbm.at[p0 + p], x_v.at[pl.ds(p * IN_F, IN_F)])
        for b in range(BPW):
            pltpu.sync_copy(bias_hbm, acc_v.at[pl.ds(b * OUT_F, OUT_F)])

        def fire(chunk, buf):
            pltpu.async_copy(
                edges_hbm.at[pl.ds(chunk * CHUNK_WORDS, CHUNK_WORDS)],
                edge_v.at[pl.ds(buf * CHUNK_WORDS, CHUNK_WORDS)], sem)

        def wait(buf):
            pltpu.make_async_copy(
                edges_hbm.at[pl.ds(0, CHUNK_WORDS)],
                edge_v.at[pl.ds(buf * CHUNK_WORDS, CHUNK_WORDS)], sem).wait()

        xp_rows = [x_v.at[pl.ds(p * IN_F, IN_F)] for p in range(PPW)]
        acc_rows = [acc_v.at[pl.ds(b * OUT_F, OUT_F)] for b in range(BPW)]

        def compute(buf):
            base = buf * CHUNK_WORDS

            def load(g):
                return (edge_v[pl.ds(base + g * LANES, LANES)],
                        edge_v[pl.ds(base + EDGE_CHUNK + g * LANES, LANES)])

            def group(g, carry):
                rc16, vi16 = carry
                c16 = rc16 & 0xFFFF
                r16 = lax.shift_right_logical(rc16, 16)
                v16 = plsc.bitcast(vi16, jnp.float32)
                xis = [plsc.load_gather(xp_rows[p], [c16]) for p in range(PPW)]
                nxt = load(g + 1)
                prods = []
                for p in range(PPW):
                    lo, hi = plsc.unpack(
                        plsc.bitcast(xis[p], jnp.bfloat16),
                        format=plsc.PackFormat.INTERLEAVED,
                        preferred_element_type=jnp.float32)
                    prods.append(lo * v16)
                    prods.append(hi * v16)
                for b in range(BPW):
                    plsc.addupdate_scatter(acc_rows[b], [r16], prods[b])
                return nxt

            lax.fori_loop(0, EDGE_CHUNK // LANES, group, load(0))

        fire(0, 0)
        fire(1, 1)

        def step(k, carry):
            for buf in range(2):
                wait(buf)
                compute(buf)
                fire(2 * k + 2 + buf, buf)
            return carry

        lax.fori_loop(0, n_chunks // 2 - 1, step, 0)
        for buf in range(2):
            wait(buf)
            compute(buf)

        for b in range(BPW):
            pltpu.sync_copy(acc_v.at[pl.ds(b * OUT_F, OUT_F)], out_hbm.at[b0 + b])

    return spmm


def kernel(input, values, bias, row_idx, col_idx):
    nnz = values.shape[0]
    n_chunks = -(-nnz // EDGE_CHUNK)
    if n_chunks % 2:
        n_chunks += 1
    pad = n_chunks * EDGE_CHUNK - nnz
    if pad:
        row_idx = jnp.concatenate([row_idx, jnp.zeros((pad,), row_idx.dtype)])
        col_idx = jnp.concatenate([col_idx, jnp.zeros((pad,), col_idx.dtype)])
        values = jnp.concatenate([values, jnp.zeros((pad,), values.dtype)])
    rc = (row_idx.astype(jnp.int32) << 16) | col_idx.astype(jnp.int32)
    vi = jax.lax.bitcast_convert_type(values, jnp.int32)
    edges = jnp.stack([rc.reshape(n_chunks, EDGE_CHUNK),
                       vi.reshape(n_chunks, EDGE_CHUNK)], axis=1).reshape(-1)
    xb = input.astype(jnp.bfloat16).reshape(B // 2, 2, IN_F)
    pairs = jnp.stack([xb[:, 0], xb[:, 1]], axis=-1)
    x_packed = jax.lax.bitcast_convert_type(pairs, jnp.int32)
    return _build(n_chunks)(x_packed, edges, bias)
